# baseline (speedup 1.0000x reference)
import functools

import jax
import jax.numpy as jnp
from jax import lax
from jax.experimental import pallas as pl
from jax.experimental.pallas import tpu as pltpu
from jax.experimental.pallas import tpu_sc as plsc

N = 10000
E = 320000
IN = 128
H = 128
OUT = 128
L = 4

NC = 2
NS = 16
NW = NC * NS
EPW = E // NW
CH = 80
NCHUNK = EPW // CH
BQ = 5
BE = CH * BQ
NBATCH = EPW // BE
WB_TILES = 10
WB_ROWS = N // WB_TILES
ZR = 40

BLK = 1000


def _wsplat(w16, i):
    idx = jnp.full((16, 1), i, jnp.int32)
    dn = lax.GatherDimensionNumbers(
        offset_dims=(), collapsed_slice_dims=(0,), start_index_map=(0,))
    return lax.gather(w16, idx, dn, (1,),
                      mode=lax.GatherScatterMode.PROMISE_IN_BOUNDS)


def _spmm_body(h_hbm, src_hbm, dst_hbm, w_hbm, out_hbm, acc,
               rows0, rows1, rows2, rows3,
               sblk0, sblk1, wblk0, wblk1, dblk0, dblk1, dblk2, dblk3,
               zbuf,
               gsem0, gsem1, gsem2, gsem3,
               ssem0, ssem1, ssem2, ssem3, ssem4,
               bsa0, bsa1, bsb0, bsb1, bsb2, bsb3, zsem):
    cid = lax.axis_index("c")
    sid = lax.axis_index("s")
    wid = cid * NS + sid
    ebase = wid * EPW

    zero16 = jnp.zeros((16,), jnp.float32)

    def _zrow(i, carry):
        for j in range(H // 16):
            zbuf[i, pl.ds(j * 16, 16)] = zero16
        return carry

    lax.fori_loop(0, ZR, _zrow, 0)

    @pl.when(sid < WB_TILES)
    def _zero_acc():
        for k in range(WB_ROWS // ZR):
            pltpu.async_copy(zbuf, acc.at[pl.ds(sid * WB_ROWS + k * ZR, ZR)],
                             zsem)
        for k in range(WB_ROWS // ZR):
            pltpu.make_async_copy(
                zbuf, acc.at[pl.ds(sid * WB_ROWS + k * ZR, ZR)], zsem).wait()

    plsc.subcore_barrier()

    rows_t = (rows0, rows1, rows2, rows3)
    gsem_t = (gsem0, gsem1, gsem2, gsem3)
    ssem_t = (ssem0, ssem1, ssem2, ssem3, ssem4)
    sblk_t = (sblk0, sblk1)
    wblk_t = (wblk0, wblk1)
    dblk_t = (dblk0, dblk1, dblk2, dblk3)
    bsa_t = (bsa0, bsa1)
    bsb_t = (bsb0, bsb1, bsb2, bsb3)

    def _issue_batch(m, sw2, d3):
        off = ebase + m * BE
        pltpu.async_copy(src_hbm.at[pl.ds(off, BE)], sblk_t[sw2], bsa_t[sw2])
        pltpu.async_copy(w_hbm.at[pl.ds(off, BE)], wblk_t[sw2], bsa_t[sw2])
        pltpu.async_copy(dst_hbm.at[pl.ds(off, BE)], dblk_t[d3], bsb_t[d3])

    def _drain_batch(sw2, d3):
        pltpu.make_async_copy(src_hbm.at[pl.ds(0, BE)], sblk_t[sw2],
                              bsa_t[sw2]).wait()
        pltpu.make_async_copy(w_hbm.at[pl.ds(0, BE)], wblk_t[sw2],
                              bsa_t[sw2]).wait()
        pltpu.make_async_copy(dst_hbm.at[pl.ds(0, BE)], dblk_t[d3],
                              bsb_t[d3]).wait()

    def _issue_gather(r4, q, sw2):
        pltpu.async_copy(h_hbm.at[sblk_t[sw2].at[pl.ds(q * CH, CH)]],
                         rows_t[r4], gsem_t[r4])

    def _wait_scatter(r4, r5, q, d3):
        pltpu.make_async_copy(
            rows_t[r4], acc.at[dblk_t[d3].at[pl.ds(q * CH, CH)]],
            ssem_t[r5]).wait()

    def _compute(r4, q, sw2):
        rows, wblk = rows_t[r4], wblk_t[sw2]
        pltpu.make_async_copy(h_hbm.at[sblk_t[sw2].at[pl.ds(0, CH)]],
                              rows, gsem_t[r4]).wait()

        def _eg(eg, carry):
            w16 = wblk[pl.ds(q * CH + eg * 16, 16)]
            for i in range(16):
                ws = _wsplat(w16, i)
                e = eg * 16 + i
                for j in range(H // 16):
                    rows[e, pl.ds(j * 16, 16)] = rows[e, pl.ds(j * 16, 16)] * ws
            return carry

        lax.fori_loop(0, CH // 16, _eg, 0)

    def _issue_scatter(r4, r5, q, d3):
        pltpu.async_copy(rows_t[r4], acc.at[dblk_t[d3].at[pl.ds(q * CH, CH)]],
                         ssem_t[r5], add=True)

    def _half(c, j5, *, wait_s=True, g2=True, batch=True):
        r4, r5 = j5 % 4, j5 % 5
        q = j5 % BQ
        if wait_s:
            jm2 = j5 - 2
            _wait_scatter((jm2) % 4, jm2 % 5, jm2 % BQ, ((jm2) // BQ) % 4)
        if g2:
            jp2 = j5 + 2
            if jp2 % BQ == 0:
                _drain_batch((jp2 // BQ) % 2, (jp2 // BQ) % 4)
            _issue_gather(jp2 % 4, jp2 % BQ, (jp2 // BQ) % 2)
        if batch and q == 0:
            m1 = c // BQ + 1
            kp1 = j5 // BQ + 1

            @pl.when(m1 < NBATCH)
            def _():
                _issue_batch(m1, kp1 % 2, kp1 % 4)
        _compute(r4, q, (j5 // BQ) % 2)
        _issue_scatter(r4, r5, q, (j5 // BQ) % 4)

    _issue_batch(0, 0, 0)
    _issue_batch(1, 1, 1)
    _drain_batch(0, 0)
    _issue_gather(0, 0, 0)
    _issue_gather(1, 1, 0)
    _half(0, 0, wait_s=False, batch=False)
    _half(1, 1, wait_s=False)

    def _block20(g, carry):
        c = g * 20 + 2
        for j in range(20):
            _half(c + j, 2 + j)
        return carry

    lax.fori_loop(0, (NCHUNK - 5) // 20, _block20, 0)
    _half(122, 122, batch=False)
    _half(123, 123, g2=False, batch=False)
    _half(124, 124, g2=False, batch=False)
    _wait_scatter(123 % 4, 123 % 5, 123 % BQ, (123 // BQ) % 4)
    _wait_scatter(124 % 4, 124 % 5, 124 % BQ, (124 // BQ) % 4)

    plsc.subcore_barrier()

    @pl.when(sid < WB_TILES)
    def _writeout():
        r0 = sid * WB_ROWS
        pltpu.sync_copy(acc.at[pl.ds(r0, WB_ROWS)],
                        out_hbm.at[cid, pl.ds(r0, WB_ROWS)])


_spmm = functools.partial(
    pl.kernel,
    out_type=jax.ShapeDtypeStruct((NC, N, H), jnp.float32),
    mesh=plsc.VectorSubcoreMesh(core_axis_name="c", subcore_axis_name="s"),
    scratch_types=(
        [pltpu.VMEM_SHARED((N, H), jnp.float32)]
        + [pltpu.VMEM((CH, H), jnp.float32) for _ in range(4)]
        + [pltpu.VMEM((BE,), jnp.int32) for _ in range(2)]
        + [pltpu.VMEM((BE,), jnp.float32) for _ in range(2)]
        + [pltpu.VMEM((BE,), jnp.int32) for _ in range(4)]
        + [pltpu.VMEM((ZR, H), jnp.float32)]
        + [pltpu.SemaphoreType.DMA for _ in range(16)]
    ),
)(_spmm_body)


_DN = (((1,), (1,)), ((), ()))


def _stage_in_body(x_ref, win_ref, bin_ref, wl0_ref, bl0_ref, t_ref,
                   hid_ref, g_ref):
    h0 = lax.dot_general(x_ref[...], win_ref[...], _DN,
                         preferred_element_type=jnp.float32) + bin_ref[...]
    hid_ref[...] = h0 * t_ref[0, 0]
    g_ref[...] = lax.dot_general(h0, wl0_ref[...], _DN,
                                 preferred_element_type=jnp.float32) + bl0_ref[...]


def _stage_mid_body(y2_ref, hid_ref, w_ref, b_ref, t_ref, hid_out_ref, g_ref):
    h = jnp.maximum(y2_ref[0] + y2_ref[1], 0.0)
    hid_out_ref[...] = hid_ref[...] + h * t_ref[0, 0]
    g_ref[...] = lax.dot_general(h, w_ref[...], _DN,
                                 preferred_element_type=jnp.float32) + b_ref[...]


def _stage_out_body(y2_ref, hid_ref, wout_ref, bout_ref, t_ref, out_ref):
    h = jnp.maximum(y2_ref[0] + y2_ref[1], 0.0)
    hid = hid_ref[...] + h * t_ref[0, 0]
    out_ref[...] = lax.dot_general(hid, wout_ref[...], _DN,
                                   preferred_element_type=jnp.float32) + bout_ref[...]


def _row_spec(d):
    return pl.BlockSpec((BLK, d), lambda i: (i, 0))


def _full_spec(shape):
    nd = len(shape)
    return pl.BlockSpec(shape, lambda i: (0,) * nd)


_stage_in = pl.pallas_call(
    _stage_in_body,
    grid=(N // BLK,),
    in_specs=[
        _row_spec(IN),
        _full_spec((H, IN)),
        _full_spec((1, H)),
        _full_spec((H, H)),
        _full_spec((1, H)),
        _full_spec((1, 1)),
    ],
    out_specs=[_row_spec(H), _row_spec(H)],
    out_shape=[jax.ShapeDtypeStruct((N, H), jnp.float32)] * 2,
)

_stage_mid = pl.pallas_call(
    _stage_mid_body,
    grid=(N // BLK,),
    in_specs=[
        pl.BlockSpec((NC, BLK, H), lambda i: (0, i, 0)),
        _row_spec(H),
        _full_spec((H, H)),
        _full_spec((1, H)),
        _full_spec((1, 1)),
    ],
    out_specs=[_row_spec(H), _row_spec(H)],
    out_shape=[jax.ShapeDtypeStruct((N, H), jnp.float32)] * 2,
)

_stage_out = pl.pallas_call(
    _stage_out_body,
    grid=(N // BLK,),
    in_specs=[
        pl.BlockSpec((NC, BLK, H), lambda i: (0, i, 0)),
        _row_spec(H),
        _full_spec((OUT, H)),
        _full_spec((1, OUT)),
        _full_spec((1, 1)),
    ],
    out_specs=_row_spec(OUT),
    out_shape=jax.ShapeDtypeStruct((N, OUT), jnp.float32),
)


def kernel(x, edge_index, edge_weight, W_in, b_in, Wl, bl, W_out, b_out, temp):
    src = edge_index[0]
    dst = edge_index[1]
    t = temp.reshape(L + 1, 1, 1)

    hid, g = _stage_in(x, W_in, b_in.reshape(1, H), Wl[0],
                       bl[0].reshape(1, H), t[0])
    for i in range(L):
        y2 = _spmm(g, src, dst, edge_weight)
        if i < L - 1:
            hid, g = _stage_mid(y2, hid, Wl[i + 1], bl[i + 1].reshape(1, H),
                                t[i + 1])
        else:
            out = _stage_out(y2, hid, W_out, b_out.reshape(1, OUT), t[L])
    return out


# speedup vs baseline: 1.0196x; 1.0196x over previous
import functools

import jax
import jax.numpy as jnp
from jax import lax
from jax.experimental import pallas as pl
from jax.experimental.pallas import tpu as pltpu
from jax.experimental.pallas import tpu_sc as plsc

N = 10000
E = 320000
IN = 128
H = 128
OUT = 128
L = 4

NC = 2
NS = 16
NW = NC * NS
EPW = E // NW
CH = 80
NCHUNK = EPW // CH
BQ = 5
BE = CH * BQ
NBATCH = EPW // BE
WB_TILES = 10
WB_ROWS = N // WB_TILES
ZR = 40

BLK = 1000


def _wsplat(w16, i):
    idx = jnp.full((16, 1), i, jnp.int32)
    dn = lax.GatherDimensionNumbers(
        offset_dims=(), collapsed_slice_dims=(0,), start_index_map=(0,))
    return lax.gather(w16, idx, dn, (1,),
                      mode=lax.GatherScatterMode.PROMISE_IN_BOUNDS)


def _spmm_body(h_hbm, src_hbm, dst_hbm, w_hbm, out_hbm, acc,
               rows0, rows1, rows2, rows3,
               sblk0, sblk1, wblk0, wblk1, dblk0, dblk1, dblk2, dblk3,
               zbuf,
               gsem0, gsem1, gsem2, gsem3,
               ssem0, ssem1, ssem2, ssem3, ssem4,
               bsa0, bsa1, bsb0, bsb1, bsb2, bsb3, zsem):
    cid = lax.axis_index("c")
    sid = lax.axis_index("s")
    wid = cid * NS + sid
    ebase = wid * EPW

    zero16 = jnp.zeros((16,), jnp.float32)

    def _zrow(i, carry):
        for j in range(H // 16):
            zbuf[i, pl.ds(j * 16, 16)] = zero16
        return carry

    lax.fori_loop(0, ZR, _zrow, 0)

    @pl.when(sid < WB_TILES)
    def _zero_acc():
        for k in range(WB_ROWS // ZR):
            pltpu.async_copy(zbuf, acc.at[pl.ds(sid * WB_ROWS + k * ZR, ZR)],
                             zsem)

    rows_t = (rows0, rows1, rows2, rows3)
    gsem_t = (gsem0, gsem1, gsem2, gsem3)
    ssem_t = (ssem0, ssem1, ssem2, ssem3, ssem4)
    sblk_t = (sblk0, sblk1)
    wblk_t = (wblk0, wblk1)
    dblk_t = (dblk0, dblk1, dblk2, dblk3)
    bsa_t = (bsa0, bsa1)
    bsb_t = (bsb0, bsb1, bsb2, bsb3)

    def _issue_batch(m, sw2, d3):
        off = ebase + m * BE
        pltpu.async_copy(src_hbm.at[pl.ds(off, BE)], sblk_t[sw2], bsa_t[sw2])
        pltpu.async_copy(w_hbm.at[pl.ds(off, BE)], wblk_t[sw2], bsa_t[sw2])
        pltpu.async_copy(dst_hbm.at[pl.ds(off, BE)], dblk_t[d3], bsb_t[d3])

    def _drain_batch(sw2, d3):
        pltpu.make_async_copy(src_hbm.at[pl.ds(0, BE)], sblk_t[sw2],
                              bsa_t[sw2]).wait()
        pltpu.make_async_copy(w_hbm.at[pl.ds(0, BE)], wblk_t[sw2],
                              bsa_t[sw2]).wait()
        pltpu.make_async_copy(dst_hbm.at[pl.ds(0, BE)], dblk_t[d3],
                              bsb_t[d3]).wait()

    def _issue_gather(r4, q, sw2):
        pltpu.async_copy(h_hbm.at[sblk_t[sw2].at[pl.ds(q * CH, CH)]],
                         rows_t[r4], gsem_t[r4])

    def _wait_scatter(r4, r5, q, d3):
        pltpu.make_async_copy(
            rows_t[r4], acc.at[dblk_t[d3].at[pl.ds(q * CH, CH)]],
            ssem_t[r5]).wait()

    def _compute(r4, q, sw2):
        rows, wblk = rows_t[r4], wblk_t[sw2]
        pltpu.make_async_copy(h_hbm.at[sblk_t[sw2].at[pl.ds(0, CH)]],
                              rows, gsem_t[r4]).wait()

        def _eg(eg, carry):
            w16 = wblk[pl.ds(q * CH + eg * 16, 16)]
            for i in range(16):
                ws = _wsplat(w16, i)
                e = eg * 16 + i
                for j in range(H // 16):
                    rows[e, pl.ds(j * 16, 16)] = rows[e, pl.ds(j * 16, 16)] * ws
            return carry

        lax.fori_loop(0, CH // 16, _eg, 0)

    def _issue_scatter(r4, r5, q, d3):
        pltpu.async_copy(rows_t[r4], acc.at[dblk_t[d3].at[pl.ds(q * CH, CH)]],
                         ssem_t[r5], add=True)

    def _half(c, j5, *, wait_s=True, g2=True, batch=True):
        r4, r5 = j5 % 4, j5 % 5
        q = j5 % BQ
        if wait_s:
            jm2 = j5 - 2
            _wait_scatter(jm2 % 4, jm2 % 5, jm2 % BQ, ((jm2) // BQ) % 4)
        if g2:
            jp2 = j5 + 2
            if jp2 % BQ == 0:
                _drain_batch((jp2 // BQ) % 2, (jp2 // BQ) % 4)
            _issue_gather(jp2 % 4, jp2 % BQ, (jp2 // BQ) % 2)
        if batch and q == 0:
            m1 = c // BQ + 1
            kp1 = j5 // BQ + 1

            @pl.when(m1 < NBATCH)
            def _():
                _issue_batch(m1, kp1 % 2, kp1 % 4)
        _compute(r4, q, (j5 // BQ) % 2)
        _issue_scatter(r4, r5, q, (j5 // BQ) % 4)

    _issue_batch(0, 0, 0)
    _issue_batch(1, 1, 1)
    _drain_batch(0, 0)
    _issue_gather(0, 0, 0)
    _issue_gather(1, 1, 0)

    @pl.when(sid < WB_TILES)
    def _zero_drain():
        for k in range(WB_ROWS // ZR):
            pltpu.make_async_copy(
                zbuf, acc.at[pl.ds(sid * WB_ROWS + k * ZR, ZR)], zsem).wait()

    plsc.subcore_barrier()
    _half(0, 0, wait_s=False, batch=False)
    _half(1, 1, wait_s=False)

    def _block20(g, carry):
        c = g * 20 + 2
        for j in range(20):
            _half(c + j, 2 + j)
        return carry

    lax.fori_loop(0, (NCHUNK - 5) // 20, _block20, 0)
    _half(122, 122, batch=False)
    _half(123, 123, g2=False, batch=False)
    _half(124, 124, g2=False, batch=False)
    _wait_scatter(123 % 4, 123 % 5, 123 % BQ, (123 // BQ) % 4)
    _wait_scatter(124 % 4, 124 % 5, 124 % BQ, (124 // BQ) % 4)

    plsc.subcore_barrier()

    @pl.when(sid < WB_TILES)
    def _writeout():
        r0 = sid * WB_ROWS
        pltpu.sync_copy(acc.at[pl.ds(r0, WB_ROWS)],
                        out_hbm.at[cid, pl.ds(r0, WB_ROWS)])


_spmm = functools.partial(
    pl.kernel,
    out_type=jax.ShapeDtypeStruct((NC, N, H), jnp.float32),
    mesh=plsc.VectorSubcoreMesh(core_axis_name="c", subcore_axis_name="s"),
    scratch_types=(
        [pltpu.VMEM_SHARED((N, H), jnp.float32)]
        + [pltpu.VMEM((CH, H), jnp.float32) for _ in range(4)]
        + [pltpu.VMEM((BE,), jnp.int32) for _ in range(2)]
        + [pltpu.VMEM((BE,), jnp.float32) for _ in range(2)]
        + [pltpu.VMEM((BE,), jnp.int32) for _ in range(4)]
        + [pltpu.VMEM((ZR, H), jnp.float32)]
        + [pltpu.SemaphoreType.DMA for _ in range(16)]
    ),
)(_spmm_body)


_DN = (((1,), (1,)), ((), ()))


def _stage_in_body(x_ref, win_ref, bin_ref, wl0_ref, bl0_ref, t_ref,
                   hid_ref, g_ref):
    h0 = lax.dot_general(x_ref[...], win_ref[...], _DN,
                         preferred_element_type=jnp.float32) + bin_ref[...]
    hid_ref[...] = h0 * t_ref[0, 0]
    g_ref[...] = lax.dot_general(h0, wl0_ref[...], _DN,
                                 preferred_element_type=jnp.float32) + bl0_ref[...]


def _stage_mid_body(y2_ref, hid_ref, w_ref, b_ref, t_ref, hid_out_ref, g_ref):
    h = jnp.maximum(y2_ref[0] + y2_ref[1], 0.0)
    hid_out_ref[...] = hid_ref[...] + h * t_ref[0, 0]
    g_ref[...] = lax.dot_general(h, w_ref[...], _DN,
                                 preferred_element_type=jnp.float32) + b_ref[...]


def _stage_out_body(y2_ref, hid_ref, wout_ref, bout_ref, t_ref, out_ref):
    h = jnp.maximum(y2_ref[0] + y2_ref[1], 0.0)
    hid = hid_ref[...] + h * t_ref[0, 0]
    out_ref[...] = lax.dot_general(hid, wout_ref[...], _DN,
                                   preferred_element_type=jnp.float32) + bout_ref[...]


def _row_spec(d):
    return pl.BlockSpec((BLK, d), lambda i: (i, 0))


def _full_spec(shape):
    nd = len(shape)
    return pl.BlockSpec(shape, lambda i: (0,) * nd)


_stage_in = pl.pallas_call(
    _stage_in_body,
    grid=(N // BLK,),
    in_specs=[
        _row_spec(IN),
        _full_spec((H, IN)),
        _full_spec((1, H)),
        _full_spec((H, H)),
        _full_spec((1, H)),
        _full_spec((1, 1)),
    ],
    out_specs=[_row_spec(H), _row_spec(H)],
    out_shape=[jax.ShapeDtypeStruct((N, H), jnp.float32)] * 2,
)

_stage_mid = pl.pallas_call(
    _stage_mid_body,
    grid=(N // BLK,),
    in_specs=[
        pl.BlockSpec((NC, BLK, H), lambda i: (0, i, 0)),
        _row_spec(H),
        _full_spec((H, H)),
        _full_spec((1, H)),
        _full_spec((1, 1)),
    ],
    out_specs=[_row_spec(H), _row_spec(H)],
    out_shape=[jax.ShapeDtypeStruct((N, H), jnp.float32)] * 2,
)

_stage_out = pl.pallas_call(
    _stage_out_body,
    grid=(N // BLK,),
    in_specs=[
        pl.BlockSpec((NC, BLK, H), lambda i: (0, i, 0)),
        _row_spec(H),
        _full_spec((OUT, H)),
        _full_spec((1, OUT)),
        _full_spec((1, 1)),
    ],
    out_specs=_row_spec(OUT),
    out_shape=jax.ShapeDtypeStruct((N, OUT), jnp.float32),
)


def kernel(x, edge_index, edge_weight, W_in, b_in, Wl, bl, W_out, b_out, temp):
    src = edge_index[0]
    dst = edge_index[1]
    t = temp.reshape(L + 1, 1, 1)

    hid, g = _stage_in(x, W_in, b_in.reshape(1, H), Wl[0],
                       bl[0].reshape(1, H), t[0])
    for i in range(L):
        y2 = _spmm(g, src, dst, edge_weight)
        if i < L - 1:
            hid, g = _stage_mid(y2, hid, Wl[i + 1], bl[i + 1].reshape(1, H),
                                t[i + 1])
        else:
            out = _stage_out(y2, hid, W_out, b_out.reshape(1, OUT), t[L])
    return out


# speedup vs baseline: 1.0197x; 1.0001x over previous
import functools

import jax
import jax.numpy as jnp
from jax import lax
from jax.experimental import pallas as pl
from jax.experimental.pallas import tpu as pltpu
from jax.experimental.pallas import tpu_sc as plsc

N = 10000
E = 320000
IN = 128
H = 128
OUT = 128
L = 4

NC = 2
NS = 16
NW = NC * NS
EPW = E // NW
CH = 80
NCHUNK = EPW // CH
BQ = 5
BE = CH * BQ
NBATCH = EPW // BE
WB_TILES = 10
WB_ROWS = N // WB_TILES
ZR = 40

BLK = 1000


def _wsplat(w16, i):
    idx = jnp.full((16, 1), i, jnp.int32)
    dn = lax.GatherDimensionNumbers(
        offset_dims=(), collapsed_slice_dims=(0,), start_index_map=(0,))
    return lax.gather(w16, idx, dn, (1,),
                      mode=lax.GatherScatterMode.PROMISE_IN_BOUNDS)


def _spmm_body(h_hbm, src_hbm, dst_hbm, w_hbm, out_hbm, acc,
               rows0, rows1, rows2, rows3,
               sblk0, sblk1, wblk0, wblk1, dblk0, dblk1, dblk2, dblk3,
               zbuf,
               gsem0, gsem1, gsem2, gsem3,
               ssem0, ssem1, ssem2, ssem3, ssem4,
               bsa0, bsa1, bsb0, bsb1, bsb2, bsb3, zsem):
    cid = lax.axis_index("c")
    sid = lax.axis_index("s")
    wid = cid * NS + sid
    ebase = wid * EPW

    zero16 = jnp.zeros((16,), jnp.float32)

    def _zrow(i, carry):
        for j in range(H // 16):
            zbuf[i, pl.ds(j * 16, 16)] = zero16
        return carry

    lax.fori_loop(0, ZR, _zrow, 0)

    @pl.when(sid < WB_TILES)
    def _zero_acc():
        for k in range(WB_ROWS // ZR):
            pltpu.async_copy(zbuf, acc.at[pl.ds(sid * WB_ROWS + k * ZR, ZR)],
                             zsem)

    rows_t = (rows0, rows1, rows2, rows3)
    gsem_t = (gsem0, gsem1, gsem2, gsem3)
    ssem_t = (ssem0, ssem1, ssem2, ssem3, ssem4)
    sblk_t = (sblk0, sblk1)
    wblk_t = (wblk0, wblk1)
    dblk_t = (dblk0, dblk1, dblk2, dblk3)
    bsa_t = (bsa0, bsa1)
    bsb_t = (bsb0, bsb1, bsb2, bsb3)

    def _issue_batch(m, sw2, d3):
        off = ebase + m * BE
        pltpu.async_copy(src_hbm.at[pl.ds(off, BE)], sblk_t[sw2], bsa_t[sw2])
        pltpu.async_copy(w_hbm.at[pl.ds(off, BE)], wblk_t[sw2], bsa_t[sw2])
        pltpu.async_copy(dst_hbm.at[pl.ds(off, BE)], dblk_t[d3], bsb_t[d3])

    def _drain_batch(sw2, d3):
        pltpu.make_async_copy(src_hbm.at[pl.ds(0, BE)], sblk_t[sw2],
                              bsa_t[sw2]).wait()
        pltpu.make_async_copy(w_hbm.at[pl.ds(0, BE)], wblk_t[sw2],
                              bsa_t[sw2]).wait()
        pltpu.make_async_copy(dst_hbm.at[pl.ds(0, BE)], dblk_t[d3],
                              bsb_t[d3]).wait()

    def _issue_gather(r4, q, sw2):
        pltpu.async_copy(h_hbm.at[sblk_t[sw2].at[pl.ds(q * CH, CH)]],
                         rows_t[r4], gsem_t[r4])

    def _wait_scatter(r4, r5, q, d3):
        pltpu.make_async_copy(
            rows_t[r4], acc.at[dblk_t[d3].at[pl.ds(q * CH, CH)]],
            ssem_t[r5]).wait()

    def _compute(r4, q, sw2):
        rows, wblk = rows_t[r4], wblk_t[sw2]
        pltpu.make_async_copy(h_hbm.at[sblk_t[sw2].at[pl.ds(0, CH)]],
                              rows, gsem_t[r4]).wait()

        def _eg(eg, carry):
            w16 = wblk[pl.ds(q * CH + eg * 16, 16)]
            for i in range(16):
                ws = _wsplat(w16, i)
                e = eg * 16 + i
                for j in range(H // 16):
                    rows[e, pl.ds(j * 16, 16)] = rows[e, pl.ds(j * 16, 16)] * ws
            return carry

        lax.fori_loop(0, CH // 16, _eg, 0)

    def _issue_scatter(r4, r5, q, d3):
        pltpu.async_copy(rows_t[r4], acc.at[dblk_t[d3].at[pl.ds(q * CH, CH)]],
                         ssem_t[r5], add=True)

    def _half(c, j5, *, wait_s=True, g2=True, batch=True):
        r4, r5 = j5 % 4, j5 % 5
        q = j5 % BQ
        if wait_s:
            jm2 = j5 - 2
            _wait_scatter(jm2 % 4, jm2 % 5, jm2 % BQ, ((jm2) // BQ) % 4)
        if g2:
            jp2 = j5 + 2
            if jp2 % BQ == 0:
                _drain_batch((jp2 // BQ) % 2, (jp2 // BQ) % 4)
            _issue_gather(jp2 % 4, jp2 % BQ, (jp2 // BQ) % 2)
        if batch and q == 0:
            m1 = c // BQ + 1
            kp1 = j5 // BQ + 1

            @pl.when(m1 < NBATCH)
            def _():
                _issue_batch(m1, kp1 % 2, kp1 % 4)
        _compute(r4, q, (j5 // BQ) % 2)
        _issue_scatter(r4, r5, q, (j5 // BQ) % 4)

    _issue_batch(0, 0, 0)
    _issue_batch(1, 1, 1)
    _drain_batch(0, 0)
    _issue_gather(0, 0, 0)
    _issue_gather(1, 1, 0)

    @pl.when(sid < WB_TILES)
    def _zero_drain():
        for k in range(WB_ROWS // ZR):
            pltpu.make_async_copy(
                zbuf, acc.at[pl.ds(sid * WB_ROWS + k * ZR, ZR)], zsem).wait()

    plsc.subcore_barrier()
    _half(0, 0, wait_s=False, batch=False)
    _half(1, 1, wait_s=False)

    def _block20(g, carry):
        c = g * 20 + 2
        for j in range(20):
            _half(c + j, 2 + j)
        return carry

    lax.fori_loop(0, (NCHUNK - 5) // 20, _block20, 0)
    _half(122, 122, batch=False)
    _half(123, 123, g2=False, batch=False)
    _half(124, 124, g2=False, batch=False)
    _wait_scatter(123 % 4, 123 % 5, 123 % BQ, (123 // BQ) % 4)
    _wait_scatter(124 % 4, 124 % 5, 124 % BQ, (124 // BQ) % 4)

    plsc.subcore_barrier()

    @pl.when(sid < NS - 1)
    def _writeout():
        r0 = sid * 624
        pltpu.sync_copy(acc.at[pl.ds(r0, 624)],
                        out_hbm.at[cid, pl.ds(r0, 624)])

    @pl.when(sid == NS - 1)
    def _writeout_last():
        pltpu.sync_copy(acc.at[pl.ds(15 * 624, N - 15 * 624)],
                        out_hbm.at[cid, pl.ds(15 * 624, N - 15 * 624)])


_spmm = functools.partial(
    pl.kernel,
    out_type=jax.ShapeDtypeStruct((NC, N, H), jnp.float32),
    mesh=plsc.VectorSubcoreMesh(core_axis_name="c", subcore_axis_name="s"),
    scratch_types=(
        [pltpu.VMEM_SHARED((N, H), jnp.float32)]
        + [pltpu.VMEM((CH, H), jnp.float32) for _ in range(4)]
        + [pltpu.VMEM((BE,), jnp.int32) for _ in range(2)]
        + [pltpu.VMEM((BE,), jnp.float32) for _ in range(2)]
        + [pltpu.VMEM((BE,), jnp.int32) for _ in range(4)]
        + [pltpu.VMEM((ZR, H), jnp.float32)]
        + [pltpu.SemaphoreType.DMA for _ in range(16)]
    ),
)(_spmm_body)


_DN = (((1,), (1,)), ((), ()))


def _stage_in_body(x_ref, win_ref, bin_ref, wl0_ref, bl0_ref, t_ref,
                   hid_ref, g_ref):
    h0 = lax.dot_general(x_ref[...], win_ref[...], _DN,
                         preferred_element_type=jnp.float32) + bin_ref[...]
    hid_ref[...] = h0 * t_ref[0, 0]
    g_ref[...] = lax.dot_general(h0, wl0_ref[...], _DN,
                                 preferred_element_type=jnp.float32) + bl0_ref[...]


def _stage_mid_body(y2_ref, hid_ref, w_ref, b_ref, t_ref, hid_out_ref, g_ref):
    h = jnp.maximum(y2_ref[0] + y2_ref[1], 0.0)
    hid_out_ref[...] = hid_ref[...] + h * t_ref[0, 0]
    g_ref[...] = lax.dot_general(h, w_ref[...], _DN,
                                 preferred_element_type=jnp.float32) + b_ref[...]


def _stage_out_body(y2_ref, hid_ref, wout_ref, bout_ref, t_ref, out_ref):
    h = jnp.maximum(y2_ref[0] + y2_ref[1], 0.0)
    hid = hid_ref[...] + h * t_ref[0, 0]
    out_ref[...] = lax.dot_general(hid, wout_ref[...], _DN,
                                   preferred_element_type=jnp.float32) + bout_ref[...]


def _row_spec(d):
    return pl.BlockSpec((BLK, d), lambda i: (i, 0))


def _full_spec(shape):
    nd = len(shape)
    return pl.BlockSpec(shape, lambda i: (0,) * nd)


_stage_in = pl.pallas_call(
    _stage_in_body,
    grid=(N // BLK,),
    in_specs=[
        _row_spec(IN),
        _full_spec((H, IN)),
        _full_spec((1, H)),
        _full_spec((H, H)),
        _full_spec((1, H)),
        _full_spec((1, 1)),
    ],
    out_specs=[_row_spec(H), _row_spec(H)],
    out_shape=[jax.ShapeDtypeStruct((N, H), jnp.float32)] * 2,
)

_stage_mid = pl.pallas_call(
    _stage_mid_body,
    grid=(N // BLK,),
    in_specs=[
        pl.BlockSpec((NC, BLK, H), lambda i: (0, i, 0)),
        _row_spec(H),
        _full_spec((H, H)),
        _full_spec((1, H)),
        _full_spec((1, 1)),
    ],
    out_specs=[_row_spec(H), _row_spec(H)],
    out_shape=[jax.ShapeDtypeStruct((N, H), jnp.float32)] * 2,
)

_stage_out = pl.pallas_call(
    _stage_out_body,
    grid=(N // BLK,),
    in_specs=[
        pl.BlockSpec((NC, BLK, H), lambda i: (0, i, 0)),
        _row_spec(H),
        _full_spec((OUT, H)),
        _full_spec((1, OUT)),
        _full_spec((1, 1)),
    ],
    out_specs=_row_spec(OUT),
    out_shape=jax.ShapeDtypeStruct((N, OUT), jnp.float32),
)


def kernel(x, edge_index, edge_weight, W_in, b_in, Wl, bl, W_out, b_out, temp):
    src = edge_index[0]
    dst = edge_index[1]
    t = temp.reshape(L + 1, 1, 1)

    hid, g = _stage_in(x, W_in, b_in.reshape(1, H), Wl[0],
                       bl[0].reshape(1, H), t[0])
    for i in range(L):
        y2 = _spmm(g, src, dst, edge_weight)
        if i < L - 1:
            hid, g = _stage_mid(y2, hid, Wl[i + 1], bl[i + 1].reshape(1, H),
                                t[i + 1])
        else:
            out = _stage_out(y2, hid, W_out, b_out.reshape(1, OUT), t[L])
    return out
